# Initial kernel scaffold; baseline (speedup 1.0000x reference)
import jax
import jax.numpy as jnp
from jax.experimental import pallas as pl


def kernel(L0, U0, pW, pB, edge_index):
    raise NotImplementedError("write your pallas kernel here")


# speedup vs baseline: 296.7283x; 296.7283x over previous
import functools

import jax
import jax.numpy as jnp
from jax import lax
from jax.experimental import pallas as pl
from jax.experimental.pallas import tpu as pltpu
from jax.experimental.pallas import tpu_sc as plsc


def _pack_body(l_ref, u_ref, o_ref):
    l = l_ref[...]
    u = u_ref[...]
    cl = 1.0 - jnp.minimum(l, u)
    cu = 1.0 - jnp.maximum(l, u)
    bl = lax.bitcast_convert_type(cl, jnp.int32)
    bu = lax.bitcast_convert_type(cu, jnp.int32)
    rl = (bl + 0x7FFF + ((bl >> 16) & 1)) >> 16
    ru = (bu + 0x7FFF + ((bu >> 16) & 1)) >> 16
    o_ref[...] = (ru << 16) | (rl & 0xFFFF)


def _combine_body(parts_ref, pb_ref, l_ref, u_ref):
    al = parts_ref[0, 0, :] + parts_ref[1, 0, :]
    au = parts_ref[0, 1, :] + parts_ref[1, 1, :]
    dg = parts_ref[0, 2, :] + parts_ref[1, 2, :]
    den = jnp.maximum(dg, 1e-6)
    pb = pb_ref[...]
    l_ref[...] = jnp.clip(pb - al / den, 0.0, 1.0)
    u_ref[...] = jnp.clip(pb - au / den, 0.0, 1.0)


def kernel(L0, U0, pW, pB, edge_index):
    N = pB.shape[0]
    E = edge_index.shape[1]
    M = pW.shape[0]
    NW = 32
    NPAD = 102400
    ZS = NPAD // 16
    SUB = 64
    CR = 25
    rows_total = E // SUB
    rows_per_tile = rows_total // NW
    nchunks = rows_per_tile // CR
    m_rows = M // SUB

    f32 = jnp.float32
    src2d = edge_index[0].reshape(rows_total, SUB)
    dst2d = edge_index[1].reshape(rows_total, SUB)
    pw2d = pW.reshape(m_rows, SUB)
    zeros = jnp.zeros((NPAD,), f32)
    pad = NPAD - N
    l0p = jnp.pad(L0, (0, pad))
    u0p = jnp.pad(U0, (0, pad))
    pbp = jnp.pad(pB, (0, pad))

    packed = pl.pallas_call(
        _pack_body,
        out_shape=jax.ShapeDtypeStruct((NPAD,), jnp.int32),
    )(l0p, u0p)

    mesh = plsc.VectorSubcoreMesh(core_axis_name="c", subcore_axis_name="s")

    @functools.partial(
        pl.kernel,
        out_type=jax.ShapeDtypeStruct((2, 3, NPAD), f32),
        mesh=mesh,
        compiler_params=pltpu.CompilerParams(use_tc_tiling_on_sc=False,
                                             needs_layout_passes=False),
        scratch_types=[
            pltpu.VMEM((N,), jnp.int32),
            pltpu.VMEM((CR, SUB), jnp.int32),
            pltpu.VMEM((CR, SUB), jnp.int32),
            pltpu.VMEM((CR, SUB), f32),
            pltpu.VMEM((CR, SUB), f32),
            pltpu.VMEM((CR, SUB), f32),
            pltpu.VMEM((SUB,), f32),
            pltpu.VMEM_SHARED((NPAD,), f32),
            pltpu.VMEM_SHARED((NPAD,), f32),
            pltpu.VMEM_SHARED((NPAD,), f32),
            pltpu.SemaphoreType.DMA,
        ],
    )
    def _edge_kernel(src_hbm, dst_hbm, pw_hbm, packed_hbm, zeros_hbm, out_hbm,
                     table_v, src_v, dst_v, pw_v, ml_v, mu_v, ones_v,
                     accl_s, accu_s, accw_s, sem):
        cid = lax.axis_index("c")
        sid = lax.axis_index("s")
        wid = sid * 2 + cid
        zoff = sid * ZS
        for acc in (accl_s, accu_s, accw_s):
            pltpu.sync_copy(zeros_hbm.at[pl.ds(zoff, ZS)],
                            acc.at[pl.ds(zoff, ZS)])
        pltpu.sync_copy(packed_hbm.at[pl.ds(0, N)], table_v)
        for j in range(SUB // 16):
            ones_v[pl.ds(j * 16, 16)] = jnp.full((16,), 1.0, f32)
        plsc.subcore_barrier()

        base_row = wid * rows_per_tile

        def run_chunks(weighted):
            @pl.loop(0, nchunks)
            def _chunk(k):
                rb = base_row + k * CR
                pltpu.make_async_copy(src_hbm.at[pl.ds(rb, CR)], src_v,
                                      sem).start()
                pltpu.make_async_copy(dst_hbm.at[pl.ds(rb, CR)], dst_v,
                                      sem).start()
                if weighted:
                    pltpu.make_async_copy(pw_hbm.at[pl.ds(rb, CR)], pw_v,
                                          sem).start()
                    pltpu.make_async_copy(pw_hbm.at[pl.ds(rb, CR)], pw_v,
                                          sem).wait()
                pltpu.make_async_copy(src_hbm.at[pl.ds(rb, CR)], src_v,
                                      sem).wait()
                pltpu.make_async_copy(dst_hbm.at[pl.ds(rb, CR)], dst_v,
                                      sem).wait()

                @pl.loop(0, CR)
                def _row(r):
                    for j in range(SUB // 16):
                        sl = (r, pl.ds(j * 16, 16))
                        g = plsc.load_gather(table_v, [src_v[sl]])
                        cl = plsc.bitcast(g << 16, f32)
                        cu = plsc.bitcast(g & jnp.int32(-65536), f32)
                        if weighted:
                            w = pw_v[sl]
                            ml_v[sl] = w * cl
                            mu_v[sl] = w * cu
                        else:
                            ml_v[sl] = cl
                            mu_v[sl] = cu

                @pl.loop(0, CR)
                def _fire(r):
                    idx = dst_v.at[r]
                    wsrc = pw_v.at[r] if weighted else ones_v
                    pltpu.make_async_copy(ml_v.at[r], accl_s.at[idx],
                                          sem).start(add=True)
                    pltpu.make_async_copy(mu_v.at[r], accu_s.at[idx],
                                          sem).start(add=True)
                    pltpu.make_async_copy(wsrc, accw_s.at[idx],
                                          sem).start(add=True)

                @pl.loop(0, CR)
                def _drain(r):
                    idx = dst_v.at[r]
                    wsrc = pw_v.at[r] if weighted else ones_v
                    pltpu.make_async_copy(ml_v.at[r], accl_s.at[idx],
                                          sem).wait()
                    pltpu.make_async_copy(mu_v.at[r], accu_s.at[idx],
                                          sem).wait()
                    pltpu.make_async_copy(wsrc, accw_s.at[idx], sem).wait()

        tile_weighted = base_row < m_rows

        @pl.when(tile_weighted)
        def _():
            run_chunks(True)

        @pl.when(jnp.logical_not(tile_weighted))
        def _():
            run_chunks(False)

        plsc.subcore_barrier()
        for j, acc in enumerate((accl_s, accu_s, accw_s)):
            pltpu.sync_copy(acc.at[pl.ds(zoff, ZS)],
                            out_hbm.at[cid, j, pl.ds(zoff, ZS)])

    parts = _edge_kernel(src2d, dst2d, pw2d, packed, zeros)

    lp, up = pl.pallas_call(
        _combine_body,
        out_shape=[jax.ShapeDtypeStruct((NPAD,), f32)] * 2,
    )(parts, pbp)
    return lp[:N], up[:N]


# speedup vs baseline: 380.2556x; 1.2815x over previous
import functools

import jax
import jax.numpy as jnp
from jax import lax
from jax.experimental import pallas as pl
from jax.experimental.pallas import tpu as pltpu
from jax.experimental.pallas import tpu_sc as plsc


def _pack_body(l_ref, u_ref, o_ref):
    l = l_ref[...]
    u = u_ref[...]
    cl = 1.0 - jnp.minimum(l, u)
    cu = 1.0 - jnp.maximum(l, u)
    bl = lax.bitcast_convert_type(cl, jnp.int32)
    bu = lax.bitcast_convert_type(cu, jnp.int32)
    rl = (bl + 0x7FFF + ((bl >> 16) & 1)) >> 16
    ru = (bu + 0x7FFF + ((bu >> 16) & 1)) >> 16
    o_ref[...] = (ru << 16) | (rl & 0xFFFF)


def _combine_body(parts_ref, pb_ref, l_ref, u_ref):
    al = parts_ref[0, 0, :] + parts_ref[1, 0, :]
    au = parts_ref[0, 1, :] + parts_ref[1, 1, :]
    dg = parts_ref[0, 2, :] + parts_ref[1, 2, :]
    den = jnp.maximum(dg, 1e-6)
    pb = pb_ref[...]
    l_ref[...] = jnp.clip(pb - al / den, 0.0, 1.0)
    u_ref[...] = jnp.clip(pb - au / den, 0.0, 1.0)


def kernel(L0, U0, pW, pB, edge_index):
    N = pB.shape[0]
    E = edge_index.shape[1]
    M = pW.shape[0]
    NW = 32
    NPAD = 102400
    ZS = NPAD // 16
    CE = 800
    edges_per_tile = E // NW
    nchunks = edges_per_tile // CE

    f32 = jnp.float32
    src1d = edge_index[0]
    dst1d = edge_index[1]
    zeros = jnp.zeros((NPAD,), f32)
    pad = NPAD - N
    l0p = jnp.pad(L0, (0, pad))
    u0p = jnp.pad(U0, (0, pad))
    pbp = jnp.pad(pB, (0, pad))

    packed = pl.pallas_call(
        _pack_body,
        out_shape=jax.ShapeDtypeStruct((NPAD,), jnp.int32),
    )(l0p, u0p)

    mesh = plsc.VectorSubcoreMesh(core_axis_name="c", subcore_axis_name="s",
                                  num_cores=2, num_subcores=16)
    ibuf2 = [pltpu.VMEM((CE,), jnp.int32)] * 2
    fbuf2 = [pltpu.VMEM((CE,), f32)] * 2

    @functools.partial(
        pl.kernel,
        out_type=jax.ShapeDtypeStruct((2, 3, NPAD), f32),
        mesh=mesh,
        compiler_params=pltpu.CompilerParams(use_tc_tiling_on_sc=False,
                                             needs_layout_passes=False),
        scratch_types=[
            pltpu.VMEM((N,), jnp.int32),
            ibuf2,
            ibuf2,
            fbuf2,
            fbuf2,
            fbuf2,
            pltpu.VMEM((CE,), f32),
            pltpu.VMEM_SHARED((NPAD,), f32),
            pltpu.VMEM_SHARED((NPAD,), f32),
            pltpu.VMEM_SHARED((NPAD,), f32),
            pltpu.SemaphoreType.DMA,
            pltpu.SemaphoreType.DMA,
        ],
    )
    def _edge_kernel(src_hbm, dst_hbm, pw_hbm, packed_hbm, zeros_hbm, out_hbm,
                     table_v, src_v, dst_v, pw_v, ml_v, mu_v, ones_v,
                     accl_s, accu_s, accw_s, sem_in, sem_sc):
        cid = lax.axis_index("c")
        sid = lax.axis_index("s")
        wid = sid * 2 + cid
        zoff = sid * ZS
        for acc in (accl_s, accu_s, accw_s):
            pltpu.sync_copy(zeros_hbm.at[pl.ds(zoff, ZS)],
                            acc.at[pl.ds(zoff, ZS)])
        pltpu.sync_copy(packed_hbm.at[pl.ds(0, N)], table_v)

        @pl.loop(0, CE // 16)
        def _init_ones(i):
            ones_v[pl.ds(i * 16, 16)] = jnp.full((16,), 1.0, f32)

        plsc.subcore_barrier()

        base = wid * edges_per_tile

        def fire_inputs(k, p, weighted, method="start"):
            eb = base + k * CE
            for hbm, v in ((src_hbm, src_v), (dst_hbm, dst_v)) + (
                    ((pw_hbm, pw_v),) if weighted else ()):
                d = pltpu.make_async_copy(hbm.at[pl.ds(eb, CE)], v[p], sem_in)
                getattr(d, method)()

        def scatter(p, weighted, method="start"):
            wsrc = pw_v[p] if weighted else ones_v
            for v, acc in ((ml_v[p], accl_s), (mu_v[p], accu_s),
                           (wsrc, accw_s)):
                d = pltpu.make_async_copy(v, acc.at[dst_v[p]], sem_sc)
                d.start(add=True) if method == "start" else d.wait()

        def compute(p, weighted):
            @pl.loop(0, CE // 16)
            def _row(i):
                sl = pl.ds(i * 16, 16)
                g = plsc.load_gather(table_v, [src_v[p][sl]])
                cl = plsc.bitcast(g << 16, f32)
                cu = plsc.bitcast(g & jnp.int32(-65536), f32)
                if weighted:
                    w = pw_v[p][sl]
                    ml_v[p][sl] = w * cl
                    mu_v[p][sl] = w * cu
                else:
                    ml_v[p][sl] = cl
                    mu_v[p][sl] = cu

        def run_chunks(weighted):
            fire_inputs(0, 0, weighted)

            @pl.loop(0, nchunks // 2)
            def _chunk2(k2):
                for p in (0, 1):
                    k = 2 * k2 + p
                    fire_inputs(k, p, weighted, "wait")
                    compute(p, weighted)
                    if p == 0:
                        @pl.when(k2 > 0)
                        def _():
                            scatter(1, weighted, "wait")
                    else:
                        scatter(0, weighted, "wait")
                    scatter(p, weighted)
                    if p == 0:
                        fire_inputs(k + 1, 1, weighted)
                    else:
                        @pl.when(k2 < nchunks // 2 - 1)
                        def _():
                            fire_inputs(2 * k2 + 2, 0, weighted)

            scatter(1, weighted, "wait")

        tile_weighted = base < M

        @pl.when(tile_weighted)
        def _():
            run_chunks(True)

        @pl.when(jnp.logical_not(tile_weighted))
        def _():
            run_chunks(False)

        plsc.subcore_barrier()
        for j, acc in enumerate((accl_s, accu_s, accw_s)):
            pltpu.sync_copy(acc.at[pl.ds(zoff, ZS)],
                            out_hbm.at[cid, j, pl.ds(zoff, ZS)])

    parts = _edge_kernel(src1d, dst1d, pW, packed, zeros)

    lp, up = pl.pallas_call(
        _combine_body,
        out_shape=[jax.ShapeDtypeStruct((NPAD,), f32)] * 2,
    )(parts, pbp)
    return lp[:N], up[:N]


# speedup vs baseline: 408.9451x; 1.0754x over previous
import functools

import jax
import jax.numpy as jnp
from jax import lax
from jax.experimental import pallas as pl
from jax.experimental.pallas import tpu as pltpu
from jax.experimental.pallas import tpu_sc as plsc


def _pack_body(l_ref, u_ref, o_ref):
    l = l_ref[...]
    u = u_ref[...]
    cl = 1.0 - jnp.minimum(l, u)
    cu = 1.0 - jnp.maximum(l, u)
    bl = lax.bitcast_convert_type(cl, jnp.int32)
    bu = lax.bitcast_convert_type(cu, jnp.int32)
    rl = (bl + 0x7FFF + ((bl >> 16) & 1)) >> 16
    ru = (bu + 0x7FFF + ((bu >> 16) & 1)) >> 16
    o_ref[...] = (ru << 16) | (rl & 0xFFFF)


def _combine_body(parts_ref, pb_ref, l_ref, u_ref):
    n = pb_ref.shape[0]
    al = parts_ref[0, pl.ds(0, n)] + parts_ref[3, pl.ds(0, n)]
    au = parts_ref[1, pl.ds(0, n)] + parts_ref[4, pl.ds(0, n)]
    dg = parts_ref[2, pl.ds(0, n)] + parts_ref[5, pl.ds(0, n)]
    den = jnp.maximum(dg, 1e-6)
    pb = pb_ref[...]
    l_ref[...] = jnp.clip(pb - al / den, 0.0, 1.0)
    u_ref[...] = jnp.clip(pb - au / den, 0.0, 1.0)


def kernel(L0, U0, pW, pB, edge_index):
    N = pB.shape[0]
    E = edge_index.shape[1]
    M = pW.shape[0]
    NW = 32
    NPAD = 102400
    ZS = NPAD // 16
    CE = 800
    edges_per_tile = E // NW
    nchunks = edges_per_tile // CE

    f32 = jnp.float32
    zeros = jnp.zeros((NPAD,), f32)
    pad = NPAD - N
    l0p = jnp.pad(L0, (0, pad))
    u0p = jnp.pad(U0, (0, pad))

    packed = pl.pallas_call(
        _pack_body,
        out_shape=jax.ShapeDtypeStruct((NPAD,), jnp.int32),
    )(l0p, u0p)

    mesh = plsc.VectorSubcoreMesh(core_axis_name="c", subcore_axis_name="s",
                                  num_cores=2, num_subcores=16)
    ibuf2 = [pltpu.VMEM((CE,), jnp.int32)] * 2
    fbuf2 = [pltpu.VMEM((CE,), f32)] * 2

    @functools.partial(
        pl.kernel,
        out_type=jax.ShapeDtypeStruct((6, NPAD), f32),
        mesh=mesh,
        compiler_params=pltpu.CompilerParams(use_tc_tiling_on_sc=False,
                                             needs_layout_passes=False),
        scratch_types=[
            pltpu.VMEM((N,), jnp.int32),
            ibuf2,
            ibuf2,
            fbuf2,
            fbuf2,
            fbuf2,
            pltpu.VMEM((CE,), f32),
            pltpu.VMEM_SHARED((NPAD,), f32),
            pltpu.VMEM_SHARED((NPAD,), f32),
            pltpu.VMEM_SHARED((NPAD,), f32),
            pltpu.SemaphoreType.DMA,
            pltpu.SemaphoreType.DMA,
        ],
    )
    def _edge_kernel(edge_hbm, pw_hbm, packed_hbm, zeros_hbm, out_hbm,
                     table_v, src_v, dst_v, pw_v, ml_v, mu_v, ones_v,
                     accl_s, accu_s, accw_s, sem_in, sem_sc):
        cid = lax.axis_index("c")
        sid = lax.axis_index("s")
        wid = sid * 2 + cid
        zoff = sid * ZS
        for acc in (accl_s, accu_s, accw_s):
            pltpu.sync_copy(zeros_hbm.at[pl.ds(zoff, ZS)],
                            acc.at[pl.ds(zoff, ZS)])
        pltpu.sync_copy(packed_hbm.at[pl.ds(0, N)], table_v)

        @pl.loop(0, CE // 16)
        def _init_ones(i):
            ones_v[pl.ds(i * 16, 16)] = jnp.full((16,), 1.0, f32)

        plsc.subcore_barrier()

        base = wid * edges_per_tile

        def fire_inputs(k, p, weighted, method="start"):
            eb = base + k * CE
            for hbm, v in ((edge_hbm.at[0], src_v), (edge_hbm.at[1], dst_v)) + (
                    ((pw_hbm, pw_v),) if weighted else ()):
                d = pltpu.make_async_copy(hbm.at[pl.ds(eb, CE)], v[p], sem_in)
                getattr(d, method)()

        def scatter(p, weighted, method="start"):
            wsrc = pw_v[p] if weighted else ones_v
            for v, acc in ((ml_v[p], accl_s), (mu_v[p], accu_s),
                           (wsrc, accw_s)):
                d = pltpu.make_async_copy(v, acc.at[dst_v[p]], sem_sc)
                d.start(add=True) if method == "start" else d.wait()

        def compute(p, weighted):
            @pl.loop(0, CE // 16)
            def _row(i):
                sl = pl.ds(i * 16, 16)
                g = plsc.load_gather(table_v, [src_v[p][sl]])
                cl = plsc.bitcast(g << 16, f32)
                cu = plsc.bitcast(g & jnp.int32(-65536), f32)
                if weighted:
                    w = pw_v[p][sl]
                    ml_v[p][sl] = w * cl
                    mu_v[p][sl] = w * cu
                else:
                    ml_v[p][sl] = cl
                    mu_v[p][sl] = cu

        def run_chunks(weighted):
            fire_inputs(0, 0, weighted)

            @pl.loop(0, nchunks // 2)
            def _chunk2(k2):
                for p in (0, 1):
                    k = 2 * k2 + p
                    fire_inputs(k, p, weighted, "wait")
                    compute(p, weighted)
                    if p == 0:
                        @pl.when(k2 > 0)
                        def _():
                            scatter(1, weighted, "wait")
                    else:
                        scatter(0, weighted, "wait")
                    scatter(p, weighted)
                    if p == 0:
                        fire_inputs(k + 1, 1, weighted)
                    else:
                        @pl.when(k2 < nchunks // 2 - 1)
                        def _():
                            fire_inputs(2 * k2 + 2, 0, weighted)

            scatter(1, weighted, "wait")

        tile_weighted = base < M

        @pl.when(tile_weighted)
        def _():
            run_chunks(True)

        @pl.when(jnp.logical_not(tile_weighted))
        def _():
            run_chunks(False)

        plsc.subcore_barrier()
        for j, acc in enumerate((accl_s, accu_s, accw_s)):
            pltpu.sync_copy(acc.at[pl.ds(zoff, ZS)],
                            out_hbm.at[cid * 3 + j, pl.ds(zoff, ZS)])

    parts = _edge_kernel(edge_index, pW, packed, zeros)

    lp, up = pl.pallas_call(
        _combine_body,
        out_shape=[jax.ShapeDtypeStruct((N,), f32)] * 2,
    )(parts, pB)
    return lp, up
